# baseline (speedup 1.0000x reference)
import functools

import jax
import jax.numpy as jnp
from jax import lax
from jax.experimental import pallas as pl
from jax.experimental.pallas import tpu as pltpu
from jax.experimental.pallas import tpu_sc as plsc

NCOMP = 16
NCLASS = 2
D = 768
NOUT = NCOMP * NCLASS
TILE = 2048
LANES = 16


def _mm_kernel(emb_ref, w_ref, b_ref, out_ref):
    out_ref[...] = (
        jnp.dot(emb_ref[...], w_ref[...], preferred_element_type=jnp.float32)
        + b_ref[...]
    )


def _all_logits(embeddings, w_full, b_full):
    B = embeddings.shape[0]
    return pl.pallas_call(
        _mm_kernel,
        grid=(B // TILE,),
        in_specs=[
            pl.BlockSpec((TILE, D), lambda i: (i, 0)),
            pl.BlockSpec((D, NOUT), lambda i: (0, 0)),
            pl.BlockSpec((1, NOUT), lambda i: (0, 0)),
        ],
        out_specs=pl.BlockSpec((TILE, NOUT), lambda i: (i, 0)),
        out_shape=jax.ShapeDtypeStruct((B, NOUT), jnp.float32),
    )(embeddings, w_full, b_full)


@functools.cache
def _make_select(B):
    info = plsc.get_sparse_core_info()
    nw = info.num_cores * info.num_subcores
    tpw = B // nw
    mesh = plsc.VectorSubcoreMesh(core_axis_name="c", subcore_axis_name="s")

    @functools.partial(
        pl.kernel,
        mesh=mesh,
        compiler_params=pltpu.CompilerParams(needs_layout_passes=False),
        out_type=jax.ShapeDtypeStruct((B * NCLASS,), jnp.float32),
        scratch_types=[
            pltpu.VMEM((tpw,), jnp.int32),
            pltpu.VMEM((tpw * NOUT,), jnp.float32),
            pltpu.VMEM((tpw * NCLASS,), jnp.float32),
        ],
    )
    def select(logits_hbm, cid_hbm, out_hbm, cid_v, log_v, out_v):
        wid = lax.axis_index("s") * info.num_cores + lax.axis_index("c")
        base = wid * tpw
        pltpu.sync_copy(cid_hbm.at[pl.ds(base, tpw)], cid_v)
        pltpu.sync_copy(logits_hbm.at[pl.ds(base * NOUT, tpw * NOUT)], log_v)

        def body(i, carry):
            rows = lax.iota(jnp.int32, LANES) + i * LANES
            src = rows * NOUT + cid_v[pl.ds(i * LANES, LANES)] * NCLASS
            dst = rows * NCLASS
            v0 = plsc.load_gather(log_v, [src])
            v1 = plsc.load_gather(log_v, [src + 1])
            plsc.store_scatter(out_v, [dst], v0)
            plsc.store_scatter(out_v, [dst + 1], v1)
            return carry

        lax.fori_loop(0, tpw // LANES, body, 0)
        pltpu.sync_copy(out_v, out_hbm.at[pl.ds(base * NCLASS, tpw * NCLASS)])

    return select


def kernel(embeddings, component_idx, W, b):
    B = embeddings.shape[0]
    cid = component_idx.astype(jnp.int32)
    w_full = jnp.transpose(W, (1, 0, 2)).reshape(D, NOUT)
    b_full = b.reshape(1, NOUT)
    logits = _all_logits(embeddings, w_full, b_full)
    out_flat = _make_select(B)(logits.reshape(B * NOUT), cid)
    return out_flat.reshape(B, NCLASS)


# speedup vs baseline: 1.7226x; 1.7226x over previous
import functools

import jax
import jax.numpy as jnp
from jax import lax
from jax.experimental import pallas as pl

NCOMP = 16
NCLASS = 2
D = 768
NOUT = NCOMP * NCLASS
TILE = 2048
KSPLIT = 2
DK = D // KSPLIT


def _mm_select_kernel(cid_ref, emb0_ref, emb1_ref, w_ref, b_ref, out_ref):
    acc = jnp.dot(
        emb0_ref[...], w_ref[0:DK, :], preferred_element_type=jnp.float32
    )
    acc += jnp.dot(
        emb1_ref[...], w_ref[DK : 2 * DK, :], preferred_element_type=jnp.float32
    )
    logits = acc + b_ref[...]
    lane = lax.broadcasted_iota(jnp.int32, (TILE, NOUT), 1)
    sel = (lane // NCLASS) == cid_ref[...]
    masked = jnp.where(sel, logits, 0.0)
    even = (lane % NCLASS) == 0
    out0 = jnp.sum(jnp.where(even, masked, 0.0), axis=1, keepdims=True)
    out1 = jnp.sum(jnp.where(even, 0.0, masked), axis=1, keepdims=True)
    out_ref[...] = jnp.concatenate([out0, out1], axis=1)


def kernel(embeddings, component_idx, W, b):
    B = embeddings.shape[0]
    cid = component_idx.astype(jnp.int32).reshape(B, 1)
    w_full = jnp.transpose(W, (1, 0, 2)).reshape(D, NOUT)
    b_full = b.reshape(1, NOUT)

    out = pl.pallas_call(
        _mm_select_kernel,
        grid=(B // TILE,),
        in_specs=[
            pl.BlockSpec((TILE, 1), lambda i: (i, 0)),
            pl.BlockSpec((TILE, DK), lambda i: (i, 0)),
            pl.BlockSpec((TILE, DK), lambda i: (i, 1)),
            pl.BlockSpec((D, NOUT), lambda i: (0, 0)),
            pl.BlockSpec((1, NOUT), lambda i: (0, 0)),
        ],
        out_specs=pl.BlockSpec((TILE, NCLASS), lambda i: (i, 0)),
        out_shape=jax.ShapeDtypeStruct((B, NCLASS), jnp.float32),
    )(cid, embeddings, embeddings, w_full, b_full)
    return out


# speedup vs baseline: 1.7540x; 1.0182x over previous
import functools

import jax
import jax.numpy as jnp
from jax import lax
from jax.experimental import pallas as pl

NCOMP = 16
NCLASS = 2
D = 768
NOUT = NCOMP * NCLASS
TILE = 4096


def _mm_select_kernel(cid_ref, emb_ref, w_ref, b_ref, out_ref):
    logits = (
        jnp.dot(emb_ref[...], w_ref[...], preferred_element_type=jnp.float32)
        + b_ref[...]
    )
    lane = lax.broadcasted_iota(jnp.int32, (TILE, NOUT), 1)
    sel = (lane // NCLASS) == cid_ref[...]
    masked = jnp.where(sel, logits, 0.0)
    even = (lane % NCLASS) == 0
    out0 = jnp.sum(jnp.where(even, masked, 0.0), axis=1, keepdims=True)
    out1 = jnp.sum(jnp.where(even, 0.0, masked), axis=1, keepdims=True)
    out_ref[...] = jnp.concatenate([out0, out1], axis=1)


def kernel(embeddings, component_idx, W, b):
    B = embeddings.shape[0]
    cid = component_idx.astype(jnp.int32).reshape(B, 1)
    w_full = jnp.transpose(W, (1, 0, 2)).reshape(D, NOUT)
    b_full = b.reshape(1, NOUT)

    out = pl.pallas_call(
        _mm_select_kernel,
        grid=(B // TILE,),
        in_specs=[
            pl.BlockSpec((TILE, 1), lambda i: (i, 0)),
            pl.BlockSpec((TILE, D), lambda i: (i, 0)),
            pl.BlockSpec((D, NOUT), lambda i: (0, 0)),
            pl.BlockSpec((1, NOUT), lambda i: (0, 0)),
        ],
        out_specs=pl.BlockSpec((TILE, NCLASS), lambda i: (i, 0)),
        out_shape=jax.ShapeDtypeStruct((B, NCLASS), jnp.float32),
    )(cid, embeddings, w_full, b_full)
    return out
